# baseline (speedup 1.0000x reference)
import functools

import jax
import jax.numpy as jnp
import numpy as np
from jax import lax
from jax.experimental import pallas as pl
from jax.experimental.pallas import tpu as pltpu
from jax.experimental.pallas import tpu_sc as plsc

VOCAB = 30522
DIM = 768
MAX_LEN = 64
BATCH = 4096
SEQ = 64

NUM_CORES = 2
NUM_SUBCORES = 16
NUM_WORKERS = NUM_CORES * NUM_SUBCORES
ROWS = BATCH * SEQ
ROWS_PER_WORKER = ROWS // NUM_WORKERS
CHUNK = 64
CHUNKS_PER_WORKER = ROWS_PER_WORKER // CHUNK
LANES = 16
COL_GROUPS = DIM // LANES


def _fixed_sincos1d(length, dim):
    pos = np.arange(length, dtype=np.float32)[:, None]
    i = np.arange(dim // 2, dtype=np.float32)[None, :]
    angle = pos / np.power(10000.0, 2.0 * i / dim)
    return np.concatenate([np.sin(angle), np.cos(angle)], axis=-1)


def _embed_kernel(x_hbm, wte_hbm, pos_hbm, out_hbm, idx_v, buf_v, pos_v, sem):
    wid = lax.axis_index("s") * NUM_CORES + lax.axis_index("c")
    base = wid * ROWS_PER_WORKER

    pltpu.sync_copy(pos_hbm, pos_v)

    def chunk_body(g, carry):
        row0 = base + g * CHUNK
        pltpu.sync_copy(x_hbm.at[pl.ds(row0, CHUNK)], idx_v)
        pltpu.async_copy(wte_hbm.at[idx_v], buf_v, sem).wait()

        def add_body(i, c2):
            r = i // COL_GROUPS
            c = (i % COL_GROUPS) * LANES
            buf_v[r, pl.ds(c, LANES)] = (
                buf_v[r, pl.ds(c, LANES)] + pos_v[r, pl.ds(c, LANES)]
            )
            return c2

        lax.fori_loop(0, CHUNK * COL_GROUPS, add_body, 0, unroll=8)

        pltpu.sync_copy(buf_v, out_hbm.at[pl.ds(row0, CHUNK)])
        return carry

    lax.fori_loop(0, CHUNKS_PER_WORKER, chunk_body, 0)


@functools.partial(jax.jit, static_argnames=())
def kernel(x, wte):
    pos = jnp.asarray(_fixed_sincos1d(MAX_LEN, DIM), dtype=jnp.float32)
    x_flat = x.reshape(ROWS).astype(jnp.int32)

    mesh = plsc.VectorSubcoreMesh(core_axis_name="c", subcore_axis_name="s")
    run = pl.kernel(
        _embed_kernel,
        mesh=mesh,
        out_type=jax.ShapeDtypeStruct((ROWS, DIM), jnp.float32),
        scratch_types=[
            pltpu.VMEM((CHUNK,), jnp.int32),
            pltpu.VMEM((CHUNK, DIM), jnp.float32),
            pltpu.VMEM((MAX_LEN, DIM), jnp.float32),
            pltpu.SemaphoreType.DMA,
        ],
    )
    out = run(x_flat, wte, pos)
    return out.reshape(BATCH, SEQ, DIM)


# speedup vs baseline: 1.2881x; 1.2881x over previous
import functools

import jax
import jax.numpy as jnp
import numpy as np
from jax import lax
from jax.experimental import pallas as pl
from jax.experimental.pallas import tpu as pltpu
from jax.experimental.pallas import tpu_sc as plsc

VOCAB = 30522
DIM = 768
MAX_LEN = 64
BATCH = 4096
SEQ = 64

NUM_CORES = 2
NUM_SUBCORES = 16
NUM_WORKERS = NUM_CORES * NUM_SUBCORES
S_PER_WORKER = SEQ // NUM_WORKERS
CHUNK = 64
CHUNKS_PER_S = BATCH // CHUNK
CHUNKS_PER_WORKER = S_PER_WORKER * CHUNKS_PER_S
LANES = 16
COL_GROUPS = DIM // LANES


def _fixed_sincos1d(length, dim):
    pos = np.arange(length, dtype=np.float32)[:, None]
    i = np.arange(dim // 2, dtype=np.float32)[None, :]
    angle = pos / np.power(10000.0, 2.0 * i / dim)
    return np.concatenate([np.sin(angle), np.cos(angle)], axis=-1)


def _embed_kernel(xt_hbm, wte_hbm, pos_hbm, out_hbm, idx_v, buf_v, pos_v, sems):
    wid = lax.axis_index("s") * NUM_CORES + lax.axis_index("c")
    s_base = wid * S_PER_WORKER

    pltpu.sync_copy(pos_hbm.at[pl.ds(s_base, S_PER_WORKER)], pos_v)

    def fill(q, par):
        t = q // CHUNKS_PER_S
        b0 = (q % CHUNKS_PER_S) * CHUNK
        pltpu.sync_copy(xt_hbm.at[s_base + t, pl.ds(b0, CHUNK)], idx_v.at[par])
        pltpu.async_copy(wte_hbm.at[idx_v.at[par]], buf_v.at[par], sems.at[par])

    def drain(q, par):
        pltpu.make_async_copy(
            wte_hbm.at[idx_v.at[par]], buf_v.at[par], sems.at[par]
        ).wait()
        t = q // CHUNKS_PER_S
        b0 = (q % CHUNKS_PER_S) * CHUNK

        def add_row(r, carry):
            for c in range(COL_GROUPS):
                sl = pl.ds(c * LANES, LANES)
                buf_v[par, r, sl] = buf_v[par, r, sl] + pos_v[t, sl]
            return carry

        lax.fori_loop(0, CHUNK, add_row, 0)
        pltpu.sync_copy(
            buf_v.at[par], out_hbm.at[pl.ds(b0, CHUNK), s_base + t]
        )

    fill(0, 0)
    fill(1, 1)

    def body(k, carry):
        q0 = 2 * k
        drain(q0, 0)

        @pl.when(q0 + 2 < CHUNKS_PER_WORKER)
        def _():
            fill(q0 + 2, 0)

        drain(q0 + 1, 1)

        @pl.when(q0 + 3 < CHUNKS_PER_WORKER)
        def _():
            fill(q0 + 3, 1)

        return carry

    lax.fori_loop(0, CHUNKS_PER_WORKER // 2, body, 0)


@functools.partial(jax.jit, static_argnames=())
def kernel(x, wte):
    pos = jnp.asarray(_fixed_sincos1d(MAX_LEN, DIM), dtype=jnp.float32)
    x_t = x.astype(jnp.int32).T

    mesh = plsc.VectorSubcoreMesh(core_axis_name="c", subcore_axis_name="s")
    run = pl.kernel(
        _embed_kernel,
        mesh=mesh,
        out_type=jax.ShapeDtypeStruct((BATCH, SEQ, DIM), jnp.float32),
        scratch_types=[
            pltpu.VMEM((2, CHUNK), jnp.int32),
            pltpu.VMEM((2, CHUNK, DIM), jnp.float32),
            pltpu.VMEM((S_PER_WORKER, DIM), jnp.float32),
            pltpu.SemaphoreType.DMA((2,)),
        ],
    )
    return run(x_t, wte, pos)


# speedup vs baseline: 3.6891x; 2.8641x over previous
import functools

import jax
import jax.numpy as jnp
import numpy as np
from jax import lax
from jax.experimental import pallas as pl
from jax.experimental.pallas import tpu as pltpu
from jax.experimental.pallas import tpu_sc as plsc

VOCAB = 30522
DIM = 768
MAX_LEN = 64
BATCH = 4096
SEQ = 64

NUM_CORES = 2
NUM_SUBCORES = 16
NUM_WORKERS = NUM_CORES * NUM_SUBCORES
S_PER_WORKER = SEQ // NUM_WORKERS
ROWS_PER_WORKER = S_PER_WORKER * BATCH
CHUNK = 32
CHUNKS_PER_S = BATCH // CHUNK
NCHUNKS = S_PER_WORKER * CHUNKS_PER_S
RING = 4
LANES = 16
COL_GROUPS = DIM // LANES


def _fixed_sincos1d(length, dim):
    pos = np.arange(length, dtype=np.float32)[:, None]
    i = np.arange(dim // 2, dtype=np.float32)[None, :]
    angle = pos / np.power(10000.0, 2.0 * i / dim)
    return np.concatenate([np.sin(angle), np.cos(angle)], axis=-1)


def _embed_kernel(xt_hbm, wte_hbm, pos_hbm, out_hbm,
                  idx_all, buf_v, pos_v, gsem, ssem):
    wid = lax.axis_index("s") * NUM_CORES + lax.axis_index("c")
    s_base = wid * S_PER_WORKER

    pltpu.sync_copy(xt_hbm.at[pl.ds(wid * ROWS_PER_WORKER, ROWS_PER_WORKER)],
                    idx_all)
    pltpu.sync_copy(pos_hbm.at[pl.ds(s_base, S_PER_WORKER)], pos_v)

    def fill(q, par):
        @pl.when(q >= RING)
        def _():
            pltpu.make_async_copy(
                buf_v.at[par], out_hbm.at[pl.ds(0, CHUNK), 0], ssem.at[par]
            ).wait()

        pltpu.make_async_copy(
            wte_hbm.at[idx_all.at[pl.ds(q * CHUNK, CHUNK)]],
            buf_v.at[par],
            gsem.at[par],
        ).start()

    def drain(q, par):
        pltpu.make_async_copy(
            wte_hbm.at[pl.ds(0, CHUNK)], buf_v.at[par], gsem.at[par]
        ).wait()
        t = q // CHUNKS_PER_S
        b0 = (q % CHUNKS_PER_S) * CHUNK

        for c in range(COL_GROUPS):
            sl = pl.ds(c * LANES, LANES)
            pv = pos_v[t, sl]

            @plsc.parallel_loop(0, CHUNK, unroll=4)
            def _(r):
                buf_v[par, r, sl] = buf_v[par, r, sl] + pv

        pltpu.make_async_copy(
            buf_v.at[par], out_hbm.at[pl.ds(b0, CHUNK), s_base + t],
            ssem.at[par],
        ).start()

    fill(0, 0)
    fill(1, 1)
    fill(2, 2)

    def body(k, carry):
        q0 = RING * k
        for j in range(RING):
            q = q0 + j
            drain(q, j)

            @pl.when(q + RING - 1 < NCHUNKS)
            def _():
                fill(q + RING - 1, (j + RING - 1) % RING)

        return carry

    lax.fori_loop(0, NCHUNKS // RING, body, 0)

    for par in range(RING):
        pltpu.make_async_copy(
            buf_v.at[par], out_hbm.at[pl.ds(0, CHUNK), 0], ssem.at[par]
        ).wait()


@functools.partial(jax.jit, static_argnames=())
def kernel(x, wte):
    pos = jnp.asarray(_fixed_sincos1d(MAX_LEN, DIM), dtype=jnp.float32)
    xt_flat = x.astype(jnp.int32).T.reshape(SEQ * BATCH)

    mesh = plsc.VectorSubcoreMesh(core_axis_name="c", subcore_axis_name="s")
    run = pl.kernel(
        _embed_kernel,
        mesh=mesh,
        out_type=jax.ShapeDtypeStruct((BATCH, SEQ, DIM), jnp.float32),
        scratch_types=[
            pltpu.VMEM((ROWS_PER_WORKER,), jnp.int32),
            pltpu.VMEM((RING, CHUNK, DIM), jnp.float32),
            pltpu.VMEM((S_PER_WORKER, DIM), jnp.float32),
            pltpu.SemaphoreType.DMA((RING,)),
            pltpu.SemaphoreType.DMA((RING,)),
        ],
    )
    return run(xt_flat, wte, pos)
